# baseline (speedup 1.0000x reference)
import functools

import jax
import jax.numpy as jnp
from jax import lax
from jax.experimental import pallas as pl
from jax.experimental.pallas import tpu as pltpu
from jax.experimental.pallas import tpu_sc as plsc

N = 8192
D = 256
RB = 1024
NB = N // RB
K = 1024

NTILES = 16
CHUNK = N // NTILES
QUOTA = 96
NCAND = NTILES * QUOTA
BISECT_ITERS = 45


NPAIR = NB * (NB + 1) // 2
PPS = 6


def _decode(t):
    i = jnp.int32(0)
    for b in range(1, NB):
        tb = b * NB - (b * (b - 1)) // 2
        i = i + jnp.where(t >= tb, jnp.int32(1), jnp.int32(0))
    tstart = i * NB - (i * (i - 1)) // 2
    j = i + (t - tstart)
    return i, j


def _tri_body(x_ref, wcol_ref, wrow_ref, mc_ref, mr_ref):
    s = pl.program_id(0)
    for u in range(PPS):
        t = s * PPS + u
        i, j = _decode(t)
        xa = x_ref[pl.ds(i * RB, RB), :]
        xb = x_ref[pl.ds(j * RB, RB), :]
        P = jax.lax.dot_general(
            xa, xb, (((1,), (1,)), ((), ())),
            preferred_element_type=jnp.float32)
        wc = wcol_ref[pl.ds(i * RB, RB), :]
        wr = wrow_ref[:, pl.ds(j * RB, RB)]
        G = P / (wc * wr)
        r = jax.lax.broadcasted_iota(jnp.int32, (RB, RB), 0)
        c = jax.lax.broadcasted_iota(jnp.int32, (RB, RB), 1)
        Gm = jnp.where((r == c) & (i == j), -jnp.inf, G)
        rowm = jnp.max(Gm, axis=1, keepdims=True)
        colm = jnp.max(Gm, axis=0, keepdims=True)
        oldc = mc_ref[pl.ds(i * RB, RB), :]
        mc_ref[pl.ds(i * RB, RB), :] = jnp.where(
            j == i, rowm, jnp.maximum(oldc, rowm))
        oldr = mr_ref[:, pl.ds(j * RB, RB)]
        mr_ref[:, pl.ds(j * RB, RB)] = jnp.where(
            i == 0, colm, jnp.maximum(oldr, colm))


def _rowmax_tri(x, w_col, w_row):
    return pl.pallas_call(
        _tri_body,
        grid=(NPAIR // PPS,),
        in_specs=[
            pl.BlockSpec((N, D), lambda s: (0, 0)),
            pl.BlockSpec((N, 1), lambda s: (0, 0)),
            pl.BlockSpec((1, N), lambda s: (0, 0)),
        ],
        out_specs=[
            pl.BlockSpec((N, 1), lambda s: (0, 0)),
            pl.BlockSpec((1, N), lambda s: (0, 0)),
        ],
        out_shape=[
            jax.ShapeDtypeStruct((N, 1), jnp.float32),
            jax.ShapeDtypeStruct((1, N), jnp.float32),
        ],
    )(x, w_col, w_row)


def _bisect_body(mc_ref, mr_ref, m_ref, vk_ref):
    m = jnp.maximum(jnp.transpose(mc_ref[...]), mr_ref[...])
    m_ref[...] = m

    def it(_, lohi):
        lo, hi = lohi
        mid = (lo + hi) * jnp.float32(0.5)
        cnt = jnp.sum((m > mid).astype(jnp.int32))
        pred = cnt < K
        return (jnp.where(pred, lo, mid), jnp.where(pred, mid, hi))

    lo, hi = lax.fori_loop(
        0, BISECT_ITERS, it, (jnp.float32(-2.0), jnp.float32(2.0)))
    for l in range(16):
        vk_ref[0, l] = hi


def _combine_bisect(mc, mr):
    return pl.pallas_call(
        _bisect_body,
        in_specs=[
            pl.BlockSpec((N, 1), lambda: (0, 0)),
            pl.BlockSpec((1, N), lambda: (0, 0)),
        ],
        out_specs=[
            pl.BlockSpec((1, N), lambda: (0, 0)),
            pl.BlockSpec(memory_space=pltpu.SMEM),
        ],
        out_shape=[
            jax.ShapeDtypeStruct((1, N), jnp.float32),
            jax.ShapeDtypeStruct((1, 16), jnp.float32),
        ],
    )(mc, mr)


def _sc_select_impl(m_hbm, vk_hbm, candv_hbm, candi_hbm, mv, vkv, clv, cli):
    cid = lax.axis_index("c")
    sid = lax.axis_index("s")

    @pl.when(cid == 0)
    def _():
        t = sid
        pltpu.sync_copy(m_hbm.at[pl.ds(t * CHUNK, CHUNK)], mv)
        pltpu.sync_copy(vk_hbm, vkv)
        for q in range((QUOTA + 16) // 16):
            clv[pl.ds(q * 16, 16)] = jnp.full((16,), -jnp.inf, jnp.float32)
            cli[pl.ds(q * 16, 16)] = jnp.full((16,), N, jnp.int32)
        vk = vkv[...]
        base = t * CHUNK
        lane = lax.iota(jnp.int32, 16)
        off = jnp.int32(0)
        for k in range(CHUNK // 16):
            v = mv[pl.ds(k * 16, 16)]
            msk = v >= vk
            key = lane + jnp.where(msk, jnp.int32(0), jnp.int32(1024))
            _, vs = plsc.sort_key_val(key, v)
            _, gs = plsc.sort_key_val(key, lane + (base + k * 16))
            offc = jnp.minimum(off, QUOTA)
            clv[pl.ds(offc, 16)] = vs
            cli[pl.ds(offc, 16)] = gs
            pc = plsc.cumsum(jnp.where(msk, jnp.int32(1), jnp.int32(0)))
            off = off + pc[15]
        pltpu.sync_copy(clv.at[pl.ds(0, QUOTA)],
                        candv_hbm.at[pl.ds(t * QUOTA, QUOTA)])
        pltpu.sync_copy(cli.at[pl.ds(0, QUOTA)],
                        candi_hbm.at[pl.ds(t * QUOTA, QUOTA)])


@functools.cache
def _sc_select_kernel():
    return pl.kernel(
        _sc_select_impl,
        out_type=[
            jax.ShapeDtypeStruct((NCAND,), jnp.float32),
            jax.ShapeDtypeStruct((NCAND,), jnp.int32),
        ],
        mesh=plsc.VectorSubcoreMesh(
            core_axis_name="c", subcore_axis_name="s"),
        compiler_params=pltpu.CompilerParams(needs_layout_passes=False),
        scratch_types=[
            pltpu.VMEM((CHUNK,), jnp.float32),
            pltpu.VMEM((16,), jnp.float32),
            pltpu.VMEM((QUOTA + 16,), jnp.float32),
            pltpu.VMEM((QUOTA + 16,), jnp.int32),
        ],
    )


def _sc_select(m, vk16):
    return _sc_select_kernel()(m, vk16)


NCB = NCAND // 128


def _rank_place_body(cvr_ref, cir_ref, vals_ref, inds_ref, rank_ref):
    cvr = cvr_ref[...]
    cir = cir_ref[...]
    for b in range(NCB):
        colv = jnp.transpose(cvr[:, b * 128:(b + 1) * 128])
        coli = jnp.transpose(cir[:, b * 128:(b + 1) * 128])
        beats = (cvr > colv) | ((cvr == colv) & (cir < coli))
        rnk = jnp.sum(beats.astype(jnp.int32), axis=1, keepdims=True)
        rank_ref[:, b * 128:(b + 1) * 128] = jnp.transpose(rnk)
    rank = rank_ref[...]
    cif = cir.astype(jnp.float32)
    for rb in range(K // 128):
        r_col = jax.lax.broadcasted_iota(jnp.int32, (128, 1), 0) + rb * 128
        hit = rank == r_col
        v = jnp.max(jnp.where(hit, cvr, -jnp.inf), axis=1, keepdims=True)
        ix = jnp.sum(jnp.where(hit, cif, 0.0), axis=1, keepdims=True)
        vals_ref[:, rb * 128:(rb + 1) * 128] = jnp.transpose(v)
        inds_ref[:, rb * 128:(rb + 1) * 128] = jnp.transpose(
            ix).astype(jnp.int32)


def _rank_place(cv_row, ci_row):
    return pl.pallas_call(
        _rank_place_body,
        in_specs=[
            pl.BlockSpec((1, NCAND), lambda: (0, 0)),
            pl.BlockSpec((1, NCAND), lambda: (0, 0)),
        ],
        out_specs=[
            pl.BlockSpec((1, K), lambda: (0, 0)),
            pl.BlockSpec((1, K), lambda: (0, 0)),
        ],
        out_shape=[
            jax.ShapeDtypeStruct((1, K), jnp.float32),
            jax.ShapeDtypeStruct((1, K), jnp.int32),
        ],
        scratch_shapes=[pltpu.VMEM((1, NCAND), jnp.int32)],
    )(cv_row, ci_row)


def kernel(x, nb_selected):
    w = jnp.sqrt(jnp.sum(x * x, axis=1, keepdims=True))
    mc, mr = _rowmax_tri(x, w, w.reshape(1, N))
    m1, vk = _combine_bisect(mc, mr)
    candv, candi = _sc_select(m1.reshape(N), vk.reshape(16))
    vals, inds = _rank_place(candv.reshape(1, NCAND), candi.reshape(1, NCAND))
    return vals.reshape(K), inds.reshape(K)


# speedup vs baseline: 1.0082x; 1.0082x over previous
import functools

import jax
import jax.numpy as jnp
from jax import lax
from jax.experimental import pallas as pl
from jax.experimental.pallas import tpu as pltpu
from jax.experimental.pallas import tpu_sc as plsc

N = 8192
D = 256
RB = 1024
NB = N // RB
K = 1024

NTILES = 16
CHUNK = N // NTILES
QUOTA = 96
NCAND = NTILES * QUOTA
BISECT_ITERS = 45


NPAIR = NB * (NB + 1) // 2
PPS = 4


def _decode(t):
    i = jnp.int32(0)
    for b in range(1, NB):
        tb = b * NB - (b * (b - 1)) // 2
        i = i + jnp.where(t >= tb, jnp.int32(1), jnp.int32(0))
    tstart = i * NB - (i * (i - 1)) // 2
    j = i + (t - tstart)
    return i, j


def _tri_body(x_ref, wcol_ref, wrow_ref, mc_ref, mr_ref):
    s = pl.program_id(0)
    for u in range(PPS):
        t = s * PPS + u
        i, j = _decode(t)
        xa = x_ref[pl.ds(i * RB, RB), :]
        xb = x_ref[pl.ds(j * RB, RB), :]
        P = jax.lax.dot_general(
            xa, xb, (((1,), (1,)), ((), ())),
            preferred_element_type=jnp.float32)
        wc = wcol_ref[pl.ds(i * RB, RB), :]
        wr = wrow_ref[:, pl.ds(j * RB, RB)]
        G = P / (wc * wr)
        r = jax.lax.broadcasted_iota(jnp.int32, (RB, RB), 0)
        c = jax.lax.broadcasted_iota(jnp.int32, (RB, RB), 1)
        Gm = jnp.where((r == c) & (i == j), -jnp.inf, G)
        rowm = jnp.max(Gm, axis=1, keepdims=True)
        colm = jnp.max(Gm, axis=0, keepdims=True)
        oldc = mc_ref[pl.ds(i * RB, RB), :]
        mc_ref[pl.ds(i * RB, RB), :] = jnp.where(
            j == i, rowm, jnp.maximum(oldc, rowm))
        oldr = mr_ref[:, pl.ds(j * RB, RB)]
        mr_ref[:, pl.ds(j * RB, RB)] = jnp.where(
            i == 0, colm, jnp.maximum(oldr, colm))


def _rowmax_tri(x, w_col, w_row):
    return pl.pallas_call(
        _tri_body,
        grid=(NPAIR // PPS,),
        in_specs=[
            pl.BlockSpec((N, D), lambda s: (0, 0)),
            pl.BlockSpec((N, 1), lambda s: (0, 0)),
            pl.BlockSpec((1, N), lambda s: (0, 0)),
        ],
        out_specs=[
            pl.BlockSpec((N, 1), lambda s: (0, 0)),
            pl.BlockSpec((1, N), lambda s: (0, 0)),
        ],
        out_shape=[
            jax.ShapeDtypeStruct((N, 1), jnp.float32),
            jax.ShapeDtypeStruct((1, N), jnp.float32),
        ],
    )(x, w_col, w_row)


def _bisect_body(mc_ref, mr_ref, m_ref, vk_ref):
    m = jnp.maximum(jnp.transpose(mc_ref[...]), mr_ref[...])
    m_ref[...] = m

    def it(_, lohi):
        lo, hi = lohi
        mid = (lo + hi) * jnp.float32(0.5)
        cnt = jnp.sum((m > mid).astype(jnp.int32))
        pred = cnt < K
        return (jnp.where(pred, lo, mid), jnp.where(pred, mid, hi))

    lo, hi = lax.fori_loop(
        0, BISECT_ITERS, it, (jnp.float32(-2.0), jnp.float32(2.0)))
    for l in range(16):
        vk_ref[0, l] = hi


def _combine_bisect(mc, mr):
    return pl.pallas_call(
        _bisect_body,
        in_specs=[
            pl.BlockSpec((N, 1), lambda: (0, 0)),
            pl.BlockSpec((1, N), lambda: (0, 0)),
        ],
        out_specs=[
            pl.BlockSpec((1, N), lambda: (0, 0)),
            pl.BlockSpec(memory_space=pltpu.SMEM),
        ],
        out_shape=[
            jax.ShapeDtypeStruct((1, N), jnp.float32),
            jax.ShapeDtypeStruct((1, 16), jnp.float32),
        ],
    )(mc, mr)


def _sc_select_impl(m_hbm, vk_hbm, candv_hbm, candi_hbm, mv, vkv, clv, cli):
    cid = lax.axis_index("c")
    sid = lax.axis_index("s")

    @pl.when(cid == 0)
    def _():
        t = sid
        pltpu.sync_copy(m_hbm.at[pl.ds(t * CHUNK, CHUNK)], mv)
        pltpu.sync_copy(vk_hbm, vkv)
        for q in range((QUOTA + 16) // 16):
            clv[pl.ds(q * 16, 16)] = jnp.full((16,), -jnp.inf, jnp.float32)
            cli[pl.ds(q * 16, 16)] = jnp.full((16,), N, jnp.int32)
        vk = vkv[...]
        base = t * CHUNK
        lane = lax.iota(jnp.int32, 16)

        def step(k, off):
            v = mv[pl.ds(k * 16, 16)]
            msk = v >= vk
            key = lane + jnp.where(msk, jnp.int32(0), jnp.int32(1024))
            _, vs = plsc.sort_key_val(key, v)
            _, gs = plsc.sort_key_val(key, lane + (base + k * 16))
            offc = jnp.minimum(off, QUOTA)
            clv[pl.ds(offc, 16)] = vs
            cli[pl.ds(offc, 16)] = gs
            pc = plsc.cumsum(jnp.where(msk, jnp.int32(1), jnp.int32(0)))
            return off + pc[15]

        lax.fori_loop(0, CHUNK // 16, step, jnp.int32(0))
        pltpu.sync_copy(clv.at[pl.ds(0, QUOTA)],
                        candv_hbm.at[pl.ds(t * QUOTA, QUOTA)])
        pltpu.sync_copy(cli.at[pl.ds(0, QUOTA)],
                        candi_hbm.at[pl.ds(t * QUOTA, QUOTA)])


@functools.cache
def _sc_select_kernel():
    return pl.kernel(
        _sc_select_impl,
        out_type=[
            jax.ShapeDtypeStruct((NCAND,), jnp.float32),
            jax.ShapeDtypeStruct((NCAND,), jnp.int32),
        ],
        mesh=plsc.VectorSubcoreMesh(
            core_axis_name="c", subcore_axis_name="s"),
        compiler_params=pltpu.CompilerParams(needs_layout_passes=False),
        scratch_types=[
            pltpu.VMEM((CHUNK,), jnp.float32),
            pltpu.VMEM((16,), jnp.float32),
            pltpu.VMEM((QUOTA + 16,), jnp.float32),
            pltpu.VMEM((QUOTA + 16,), jnp.int32),
        ],
    )


def _sc_select(m, vk16):
    return _sc_select_kernel()(m, vk16)


NCB = NCAND // 128


def _rank_place_body(cvr_ref, cir_ref, vals_ref, inds_ref, rank_ref):
    cvr = cvr_ref[...]
    cir = cir_ref[...]
    for b in range(NCB):
        colv = jnp.transpose(cvr[:, b * 128:(b + 1) * 128])
        coli = jnp.transpose(cir[:, b * 128:(b + 1) * 128])
        beats = (cvr > colv) | ((cvr == colv) & (cir < coli))
        rnk = jnp.sum(beats.astype(jnp.int32), axis=1, keepdims=True)
        rank_ref[:, b * 128:(b + 1) * 128] = jnp.transpose(rnk)
    rank = rank_ref[...]
    cif = cir.astype(jnp.float32)
    for rb in range(K // 128):
        r_col = jax.lax.broadcasted_iota(jnp.int32, (128, 1), 0) + rb * 128
        hit = rank == r_col
        v = jnp.max(jnp.where(hit, cvr, -jnp.inf), axis=1, keepdims=True)
        ix = jnp.sum(jnp.where(hit, cif, 0.0), axis=1, keepdims=True)
        vals_ref[:, rb * 128:(rb + 1) * 128] = jnp.transpose(v)
        inds_ref[:, rb * 128:(rb + 1) * 128] = jnp.transpose(
            ix).astype(jnp.int32)


def _rank_place(cv_row, ci_row):
    return pl.pallas_call(
        _rank_place_body,
        in_specs=[
            pl.BlockSpec((1, NCAND), lambda: (0, 0)),
            pl.BlockSpec((1, NCAND), lambda: (0, 0)),
        ],
        out_specs=[
            pl.BlockSpec((1, K), lambda: (0, 0)),
            pl.BlockSpec((1, K), lambda: (0, 0)),
        ],
        out_shape=[
            jax.ShapeDtypeStruct((1, K), jnp.float32),
            jax.ShapeDtypeStruct((1, K), jnp.int32),
        ],
        scratch_shapes=[pltpu.VMEM((1, NCAND), jnp.int32)],
    )(cv_row, ci_row)


def kernel(x, nb_selected):
    w = jnp.sqrt(jnp.sum(x * x, axis=1, keepdims=True))
    mc, mr = _rowmax_tri(x, w, w.reshape(1, N))
    m1, vk = _combine_bisect(mc, mr)
    candv, candi = _sc_select(m1.reshape(N), vk.reshape(16))
    vals, inds = _rank_place(candv.reshape(1, NCAND), candi.reshape(1, NCAND))
    return vals.reshape(K), inds.reshape(K)
